# baseline (speedup 1.0000x reference)
import jax
import jax.numpy as jnp
import numpy as np
from jax.experimental import pallas as pl

N = 10000
E = 320000
NS, NV = 128, 16
ES, EV = 16, 1
EPS = 1e-4
LN_EPS = 1e-5


def _ident(x):
    return x


def _gvp(s, V, p, act_s, act_v):
    Wh, Wmu, Wm, bm = p
    Vh = jnp.einsum('hv,nvc->nhc', Wh, V)
    Vmu = jnp.einsum('mh,nhc->nmc', Wmu, Vh)
    sh = jnp.clip(jnp.linalg.norm(Vh, axis=-1), EPS)
    shn = jnp.concatenate([s, sh], axis=-1)
    sm = shn @ Wm.T + bm
    sd = act_s(sm)
    vmu = jnp.clip(jnp.linalg.norm(Vmu, axis=-1, keepdims=True), EPS)
    Vd = act_v(vmu) * Vmu
    return sd, Vd


def _ln(s, g, b):
    mu = jnp.mean(s, axis=-1, keepdims=True)
    var = jnp.mean((s - mu) ** 2, axis=-1, keepdims=True)
    return (s - mu) / jnp.sqrt(var + LN_EPS) * g + b


def _vnorm(V):
    n = jnp.linalg.norm(V.reshape(V.shape[0], -1), axis=-1, keepdims=True) / np.sqrt(NV)
    n = jnp.clip(n, np.sqrt(LN_EPS))
    return V / n[..., None]


def _copy_kernel(x_ref, o_ref):
    o_ref[...] = x_ref[...]


def kernel(s, V, edge_index, edge_s, edge_V, params):
    src = edge_index[0]
    dst = edge_index[1]
    sc = jnp.concatenate([s[dst], s[src], edge_s], axis=-1)
    Vc = jnp.concatenate([V[dst], V[src], edge_V], axis=-2)
    ms, mV = _gvp(sc, Vc, params['e1'], jax.nn.relu, jax.nn.sigmoid)
    ms, mV = _gvp(ms, mV, params['e2'], jax.nn.relu, jax.nn.sigmoid)
    ms, mV = _gvp(ms, mV, params['e3'], _ident, _ident)
    att, _ = _gvp(ms, mV, params['att'], jax.nn.sigmoid, _ident)
    ms = att * ms
    mV = att[..., None] * mV
    msg = jnp.concatenate([ms, mV.reshape(mV.shape[0], -1)], axis=-1)
    agg = jax.ops.segment_sum(msg, dst, num_segments=s.shape[0])
    s_agg = agg[:, :NS]
    V_agg = agg[:, NS:].reshape(-1, NV, 3)
    s1 = _ln(s + s_agg, params['ln1_g'], params['ln1_b'])
    V1 = _vnorm(V + V_agg)
    fs, fV = _gvp(s1, V1, params['f1'], jax.nn.relu, jax.nn.sigmoid)
    fs, fV = _gvp(fs, fV, params['f2'], _ident, _ident)
    s2 = _ln(s1 + fs, params['ln2_g'], params['ln2_b'])
    V2 = _vnorm(V1 + fV)
    s2 = pl.pallas_call(
        _copy_kernel,
        out_shape=jax.ShapeDtypeStruct(s2.shape, s2.dtype),
    )(s2)
    return s2, V2


# speedup vs baseline: 1.0737x; 1.0737x over previous
import functools

import jax
import jax.numpy as jnp
import numpy as np
from jax.experimental import pallas as pl
from jax.experimental.pallas import tpu as pltpu

N = 10000
E = 320000
NS, NV = 128, 16
ES, EV = 16, 1
EPS = 1e-4
LN_EPS = 1e-5

EBLK = 2000
NBLK = 1000


def _edge_kernel(sd, ss, es, vd, vs, ev,
                 w1hd, w1hs, w1he, w1mu, w1md, w1ms, w1me, w1mh, b1,
                 w2h, w2mu, w2ms, w2mh, b2,
                 w3h, w3mu, w3ms, w3mh, b3,
                 wah, wams, wamh, ba,
                 msg):
    vh = [vd[:, 16 * c:16 * (c + 1)] @ w1hd[...]
          + vs[:, 16 * c:16 * (c + 1)] @ w1hs[...]
          + ev[:, c:c + 1] * w1he[...]
          for c in range(3)]
    sh = jnp.maximum(jnp.sqrt(vh[0] ** 2 + vh[1] ** 2 + vh[2] ** 2), EPS)
    sm = (sd[...] @ w1md[...] + ss[...] @ w1ms[...]
          + es[...] @ w1me[...] + sh @ w1mh[...] + b1[...])
    m_s = jnp.maximum(sm, 0.0)
    vmu = [vh[c] @ w1mu[...] for c in range(3)]
    nmu = jnp.maximum(jnp.sqrt(vmu[0] ** 2 + vmu[1] ** 2 + vmu[2] ** 2), EPS)
    gate = jax.nn.sigmoid(nmu)
    m_v = [gate * vmu[c] for c in range(3)]

    vh = [m_v[c] @ w2h[...] for c in range(3)]
    sh = jnp.maximum(jnp.sqrt(vh[0] ** 2 + vh[1] ** 2 + vh[2] ** 2), EPS)
    sm = m_s @ w2ms[...] + sh @ w2mh[...] + b2[...]
    m_s = jnp.maximum(sm, 0.0)
    vmu = [vh[c] @ w2mu[...] for c in range(3)]
    nmu = jnp.maximum(jnp.sqrt(vmu[0] ** 2 + vmu[1] ** 2 + vmu[2] ** 2), EPS)
    gate = jax.nn.sigmoid(nmu)
    m_v = [gate * vmu[c] for c in range(3)]

    vh = [m_v[c] @ w3h[...] for c in range(3)]
    sh = jnp.maximum(jnp.sqrt(vh[0] ** 2 + vh[1] ** 2 + vh[2] ** 2), EPS)
    m_s = m_s @ w3ms[...] + sh @ w3mh[...] + b3[...]
    vmu = [vh[c] @ w3mu[...] for c in range(3)]
    nmu = jnp.maximum(jnp.sqrt(vmu[0] ** 2 + vmu[1] ** 2 + vmu[2] ** 2), EPS)
    m_v = [nmu * vmu[c] for c in range(3)]

    vh = [m_v[c] @ wah[...] for c in range(3)]
    sh = jnp.maximum(jnp.sqrt(vh[0] ** 2 + vh[1] ** 2 + vh[2] ** 2), EPS)
    att = jax.nn.sigmoid(m_s @ wams[...] + sh @ wamh[...] + ba[...])

    msg[:, 0:NS] = att * m_s
    for c in range(3):
        msg[:, NS + 16 * c:NS + 16 * (c + 1)] = att * m_v[c]


def _node_kernel(s, v, a0,
                 ln1g, ln1b, ln2g, ln2b,
                 w1h, w1mu, w1ms, w1mh, b1,
                 w2h, w2mu, w2ms, w2mh, b2,
                 s_out, v_out):
    s_in = s[...] + a0[:, 0:NS]
    v_in = [v[:, 16 * c:16 * (c + 1)] + a0[:, NS + 16 * c:NS + 16 * (c + 1)]
            for c in range(3)]

    mu = jnp.mean(s_in, axis=-1, keepdims=True)
    var = jnp.mean((s_in - mu) ** 2, axis=-1, keepdims=True)
    s1 = (s_in - mu) * jax.lax.rsqrt(var + LN_EPS) * ln1g[...] + ln1b[...]
    nrm = (v_in[0] ** 2 + v_in[1] ** 2 + v_in[2] ** 2)
    nrm = jnp.sqrt(jnp.sum(nrm, axis=-1, keepdims=True) / NV)
    nrm = jnp.maximum(nrm, np.float32(np.sqrt(LN_EPS)))
    v1 = [v_in[c] / nrm for c in range(3)]

    vh = [v1[c] @ w1h[...] for c in range(3)]
    sh = jnp.maximum(jnp.sqrt(vh[0] ** 2 + vh[1] ** 2 + vh[2] ** 2), EPS)
    sm = s1 @ w1ms[...] + sh @ w1mh[...] + b1[...]
    f_s = jnp.maximum(sm, 0.0)
    vmu = [vh[c] @ w1mu[...] for c in range(3)]
    nmu = jnp.maximum(jnp.sqrt(vmu[0] ** 2 + vmu[1] ** 2 + vmu[2] ** 2), EPS)
    gate = jax.nn.sigmoid(nmu)
    f_v = [gate * vmu[c] for c in range(3)]

    vh = [f_v[c] @ w2h[...] for c in range(3)]
    sh = jnp.maximum(jnp.sqrt(vh[0] ** 2 + vh[1] ** 2 + vh[2] ** 2), EPS)
    f_s = f_s @ w2ms[...] + sh @ w2mh[...] + b2[...]
    vmu = [vh[c] @ w2mu[...] for c in range(3)]
    nmu = jnp.maximum(jnp.sqrt(vmu[0] ** 2 + vmu[1] ** 2 + vmu[2] ** 2), EPS)
    f_v = [nmu * vmu[c] for c in range(3)]

    s_in = s1 + f_s
    mu = jnp.mean(s_in, axis=-1, keepdims=True)
    var = jnp.mean((s_in - mu) ** 2, axis=-1, keepdims=True)
    s_out[...] = (s_in - mu) * jax.lax.rsqrt(var + LN_EPS) * ln2g[...] + ln2b[...]

    v_in = [v1[c] + f_v[c] for c in range(3)]
    nrm = (v_in[0] ** 2 + v_in[1] ** 2 + v_in[2] ** 2)
    nrm = jnp.sqrt(jnp.sum(nrm, axis=-1, keepdims=True) / NV)
    nrm = jnp.maximum(nrm, np.float32(np.sqrt(LN_EPS)))
    for c in range(3):
        v_out[:, 16 * c:16 * (c + 1)] = v_in[c] / nrm


def _full(shape):
    return pl.BlockSpec(shape, lambda i: (0,) * len(shape))


def _rows(blk, cols):
    return pl.BlockSpec((blk, cols), lambda i: (i, 0))


def _edge_weights(params):
    w1h, w1mu, w1m, b1 = params['e1']
    w2h, w2mu, w2m, b2 = params['e2']
    w3h, w3mu, w3m, b3 = params['e3']
    wah, _, wam, ba = params['att']
    h1 = w1h.shape[0]
    return (
        jnp.asarray(w1h[:, 0:NV].T),
        jnp.asarray(w1h[:, NV:2 * NV].T),
        jnp.asarray(w1h[:, 2 * NV:].T),
        jnp.asarray(w1mu.T),
        jnp.asarray(w1m[:, 0:NS].T),
        jnp.asarray(w1m[:, NS:2 * NS].T),
        jnp.asarray(w1m[:, 2 * NS:2 * NS + ES].T),
        jnp.asarray(w1m[:, 2 * NS + ES:].T),
        b1.reshape(1, NS),
        jnp.asarray(w2h.T), jnp.asarray(w2mu.T),
        jnp.asarray(w2m[:, 0:NS].T), jnp.asarray(w2m[:, NS:].T), b2.reshape(1, NS),
        jnp.asarray(w3h.T), jnp.asarray(w3mu.T),
        jnp.asarray(w3m[:, 0:NS].T), jnp.asarray(w3m[:, NS:].T), b3.reshape(1, NS),
        jnp.asarray(wah.T),
        jnp.asarray(wam[:, 0:NS].T), jnp.asarray(wam[:, NS:].T), ba.reshape(1, 1),
    ), h1


def _edge_compute(sd, ss, es, vd, vs, ev, params):
    wts, h1 = _edge_weights(params)
    w_specs = [_full(w.shape) for w in wts]
    grid = E // EBLK
    return pl.pallas_call(
        _edge_kernel,
        grid=(grid,),
        in_specs=[_rows(EBLK, NS), _rows(EBLK, NS), _rows(EBLK, ES),
                  _rows(EBLK, 3 * NV), _rows(EBLK, 3 * NV), _rows(EBLK, 3),
                  *w_specs],
        out_specs=_rows(EBLK, NS + 3 * NV),
        out_shape=jax.ShapeDtypeStruct((E, NS + 3 * NV), jnp.float32),
    )(sd, ss, es, vd, vs, ev, *wts)


def _node_compute(s, v48, agg, params):
    w1h, w1mu, w1m, b1 = params['f1']
    w2h, w2mu, w2m, b2 = params['f2']
    wts = (
        params['ln1_g'].reshape(1, NS), params['ln1_b'].reshape(1, NS),
        params['ln2_g'].reshape(1, NS), params['ln2_b'].reshape(1, NS),
        jnp.asarray(w1h.T), jnp.asarray(w1mu.T),
        jnp.asarray(w1m[:, 0:NS].T), jnp.asarray(w1m[:, NS:].T), b1.reshape(1, NS),
        jnp.asarray(w2h.T), jnp.asarray(w2mu.T),
        jnp.asarray(w2m[:, 0:NS].T), jnp.asarray(w2m[:, NS:].T), b2.reshape(1, NS),
    )
    w_specs = [_full(w.shape) for w in wts]
    grid = N // NBLK
    return pl.pallas_call(
        _node_kernel,
        grid=(grid,),
        in_specs=[_rows(NBLK, NS), _rows(NBLK, 3 * NV), _rows(NBLK, NS + 3 * NV),
                  *w_specs],
        out_specs=[_rows(NBLK, NS), _rows(NBLK, 3 * NV)],
        out_shape=[jax.ShapeDtypeStruct((N, NS), jnp.float32),
                   jax.ShapeDtypeStruct((N, 3 * NV), jnp.float32)],
    )(s, v48, agg, *wts)


def kernel(s, V, edge_index, edge_s, edge_V, params):
    src = edge_index[0]
    dst = edge_index[1]
    v48 = V.transpose(0, 2, 1).reshape(N, 3 * NV)
    ev3 = edge_V.reshape(E, 3)

    sd = s[dst]
    ss = s[src]
    vd = v48[dst]
    vs = v48[src]

    msg = _edge_compute(sd, ss, edge_s, vd, vs, ev3, params)
    agg = jax.ops.segment_sum(msg, dst, num_segments=N)

    s2, v2 = _node_compute(s, v48, agg, params)
    return s2, v2.reshape(N, 3, NV).transpose(0, 2, 1)


# speedup vs baseline: 2.4681x; 2.2988x over previous
import functools

import jax
import jax.numpy as jnp
import numpy as np
from jax import lax
from jax.experimental import pallas as pl
from jax.experimental.pallas import tpu as pltpu
from jax.experimental.pallas import tpu_sc as plsc

N = 10000
E = 320000
NS, NV = 128, 16
ES, EV = 16, 1
EPS = 1e-4
LN_EPS = 1e-5

D = NS + 3 * NV
EBLK = 2000
NBLK = 1000

_NW = 32
_EPW = E // _NW
_GC = 80
_NCH = _EPW // _GC


DP = 256


def _gather_body(tab, dst2, src2, gd, gs, idxd, idxs, bufd, bufs, sem1, sem2):
    cid = lax.axis_index("c")
    sid = lax.axis_index("s")
    wid = sid * 2 + cid
    pltpu.sync_copy(dst2.at[wid], idxd)
    pltpu.sync_copy(src2.at[wid], idxs)

    def step(j, carry):
        eb = wid * _EPW + j * _GC
        cp1 = pltpu.async_copy(tab.at[idxd.at[j]], bufd, sem1)
        cp2 = pltpu.async_copy(tab.at[idxs.at[j]], bufs, sem2)
        cp1.wait()
        cp2.wait()
        pltpu.sync_copy(bufd, gd.at[pl.ds(eb, _GC)])
        pltpu.sync_copy(bufs, gs.at[pl.ds(eb, _GC)])
        return carry

    lax.fori_loop(0, _NCH, step, 0)


def _sc_gather(tab, dst2, src2):
    f = pl.kernel(
        _gather_body,
        out_type=[jax.ShapeDtypeStruct((E, DP), jnp.float32),
                  jax.ShapeDtypeStruct((E, DP), jnp.float32)],
        mesh=plsc.VectorSubcoreMesh(core_axis_name="c", subcore_axis_name="s"),
        scratch_types=[
            pltpu.VMEM((_NCH, _GC), jnp.int32),
            pltpu.VMEM((_NCH, _GC), jnp.int32),
            pltpu.VMEM((_GC, DP), jnp.float32),
            pltpu.VMEM((_GC, DP), jnp.float32),
            pltpu.SemaphoreType.DMA,
            pltpu.SemaphoreType.DMA,
        ],
    )
    return f(tab, dst2, src2)


_SNCH = (E // 16) // _GC


def _scatter_body(msg_s, msg_v, dst2, zeros, out_s, out_v, idx, buf, acc):
    cid = lax.axis_index("c")
    sid = lax.axis_index("s")

    @pl.when(sid == 0)
    def _init():
        pltpu.sync_copy(zeros, acc)

    pltpu.sync_copy(dst2.at[sid], idx)
    plsc.subcore_barrier()

    @pl.when(cid == 0)
    def _scat_s():
        def step(j, carry):
            eb = sid * (E // 16) + j * _GC
            pltpu.sync_copy(msg_s.at[pl.ds(eb, _GC)], buf)
            pltpu.sync_copy(buf, acc.at[idx.at[j]], add=True)
            return carry
        lax.fori_loop(0, _SNCH, step, 0)

    @pl.when(cid == 1)
    def _scat_v():
        def step(j, carry):
            eb = sid * (E // 16) + j * _GC
            pltpu.sync_copy(msg_v.at[pl.ds(eb, _GC)], buf)
            pltpu.sync_copy(buf, acc.at[idx.at[j]], add=True)
            return carry
        lax.fori_loop(0, _SNCH, step, 0)

    plsc.subcore_barrier()

    @pl.when((sid == 0) & (cid == 0))
    def _outs():
        pltpu.sync_copy(acc, out_s)

    @pl.when((sid == 0) & (cid == 1))
    def _outv():
        pltpu.sync_copy(acc, out_v)


def _sc_scatter(msg_s, msg_v, dst2, zeros):
    f = pl.kernel(
        _scatter_body,
        out_type=[jax.ShapeDtypeStruct((N, NS), jnp.float32),
                  jax.ShapeDtypeStruct((N, NS), jnp.float32)],
        mesh=plsc.VectorSubcoreMesh(core_axis_name="c", subcore_axis_name="s"),
        scratch_types=[
            pltpu.VMEM((_SNCH, _GC), jnp.int32),
            pltpu.VMEM((_GC, NS), jnp.float32),
            pltpu.VMEM_SHARED((N, NS), jnp.float32),
        ],
    )
    return f(msg_s, msg_v, dst2, zeros)


def _edge_kernel(td, ts, es, ev,
                 w1hd, w1hs, w1he, w1mu, w1md, w1ms, w1me, w1mh, b1,
                 w2h, w2mu, w2ms, w2mh, b2,
                 w3h, w3mu, w3ms, w3mh, b3,
                 wah, wams, wamh, ba,
                 msg_s, msg_v):
    sd = td[:, 0:NS]
    ss = ts[:, 0:NS]
    vh = [td[:, NS + 16 * c:NS + 16 * (c + 1)] @ w1hd[...]
          + ts[:, NS + 16 * c:NS + 16 * (c + 1)] @ w1hs[...]
          + ev[:, c:c + 1] * w1he[...]
          for c in range(3)]
    sh = jnp.maximum(jnp.sqrt(vh[0] ** 2 + vh[1] ** 2 + vh[2] ** 2), EPS)
    sm = (sd @ w1md[...] + ss @ w1ms[...]
          + es[...] @ w1me[...] + sh @ w1mh[...] + b1[...])
    m_s = jnp.maximum(sm, 0.0)
    vmu = [vh[c] @ w1mu[...] for c in range(3)]
    nmu = jnp.maximum(jnp.sqrt(vmu[0] ** 2 + vmu[1] ** 2 + vmu[2] ** 2), EPS)
    gate = jax.nn.sigmoid(nmu)
    m_v = [gate * vmu[c] for c in range(3)]

    vh = [m_v[c] @ w2h[...] for c in range(3)]
    sh = jnp.maximum(jnp.sqrt(vh[0] ** 2 + vh[1] ** 2 + vh[2] ** 2), EPS)
    sm = m_s @ w2ms[...] + sh @ w2mh[...] + b2[...]
    m_s = jnp.maximum(sm, 0.0)
    vmu = [vh[c] @ w2mu[...] for c in range(3)]
    nmu = jnp.maximum(jnp.sqrt(vmu[0] ** 2 + vmu[1] ** 2 + vmu[2] ** 2), EPS)
    gate = jax.nn.sigmoid(nmu)
    m_v = [gate * vmu[c] for c in range(3)]

    vh = [m_v[c] @ w3h[...] for c in range(3)]
    sh = jnp.maximum(jnp.sqrt(vh[0] ** 2 + vh[1] ** 2 + vh[2] ** 2), EPS)
    m_s = m_s @ w3ms[...] + sh @ w3mh[...] + b3[...]
    vmu = [vh[c] @ w3mu[...] for c in range(3)]
    nmu = jnp.maximum(jnp.sqrt(vmu[0] ** 2 + vmu[1] ** 2 + vmu[2] ** 2), EPS)
    m_v = [nmu * vmu[c] for c in range(3)]

    vh = [m_v[c] @ wah[...] for c in range(3)]
    sh = jnp.maximum(jnp.sqrt(vh[0] ** 2 + vh[1] ** 2 + vh[2] ** 2), EPS)
    att = jax.nn.sigmoid(m_s @ wams[...] + sh @ wamh[...] + ba[...])

    msg_s[...] = att * m_s
    for c in range(3):
        msg_v[:, 16 * c:16 * (c + 1)] = att * m_v[c]
    msg_v[:, 48:NS] = jnp.zeros((msg_v.shape[0], NS - 48), jnp.float32)


def _full(shape):
    return pl.BlockSpec(shape, lambda i: (0,) * len(shape))


def _rows(blk, cols):
    return pl.BlockSpec((blk, cols), lambda i: (i, 0))


def _edge_weights(params):
    w1h, w1mu, w1m, b1 = params['e1']
    w2h, w2mu, w2m, b2 = params['e2']
    w3h, w3mu, w3m, b3 = params['e3']
    wah, _, wam, ba = params['att']
    return (
        jnp.asarray(w1h[:, 0:NV].T),
        jnp.asarray(w1h[:, NV:2 * NV].T),
        jnp.asarray(w1h[:, 2 * NV:].T),
        jnp.asarray(w1mu.T),
        jnp.asarray(w1m[:, 0:NS].T),
        jnp.asarray(w1m[:, NS:2 * NS].T),
        jnp.asarray(w1m[:, 2 * NS:2 * NS + ES].T),
        jnp.asarray(w1m[:, 2 * NS + ES:].T),
        b1.reshape(1, NS),
        jnp.asarray(w2h.T), jnp.asarray(w2mu.T),
        jnp.asarray(w2m[:, 0:NS].T), jnp.asarray(w2m[:, NS:].T), b2.reshape(1, NS),
        jnp.asarray(w3h.T), jnp.asarray(w3mu.T),
        jnp.asarray(w3m[:, 0:NS].T), jnp.asarray(w3m[:, NS:].T), b3.reshape(1, NS),
        jnp.asarray(wah.T),
        jnp.asarray(wam[:, 0:NS].T), jnp.asarray(wam[:, NS:].T), ba.reshape(1, 1),
    )


def _edge_compute(td, ts, es, ev3, params):
    wts = _edge_weights(params)
    w_specs = [_full(w.shape) for w in wts]
    return pl.pallas_call(
        _edge_kernel,
        grid=(E // EBLK,),
        in_specs=[_rows(EBLK, DP), _rows(EBLK, DP), _rows(EBLK, ES),
                  _rows(EBLK, 3), *w_specs],
        out_specs=[_rows(EBLK, NS), _rows(EBLK, NS)],
        out_shape=[jax.ShapeDtypeStruct((E, NS), jnp.float32),
                   jax.ShapeDtypeStruct((E, NS), jnp.float32)],
    )(td, ts, es, ev3, *wts)


def _node_kernel(s, v, a0, a1,
                 ln1g, ln1b, ln2g, ln2b,
                 w1h, w1mu, w1ms, w1mh, b1,
                 w2h, w2mu, w2ms, w2mh, b2,
                 s_out, v_out):
    s_in = s[...] + a0[...]
    v_in = [v[:, 16 * c:16 * (c + 1)] + a1[:, 16 * c:16 * (c + 1)]
            for c in range(3)]

    mu = jnp.mean(s_in, axis=-1, keepdims=True)
    var = jnp.mean((s_in - mu) ** 2, axis=-1, keepdims=True)
    s1 = (s_in - mu) * lax.rsqrt(var + LN_EPS) * ln1g[...] + ln1b[...]
    nrm = (v_in[0] ** 2 + v_in[1] ** 2 + v_in[2] ** 2)
    nrm = jnp.sqrt(jnp.sum(nrm, axis=-1, keepdims=True) / NV)
    nrm = jnp.maximum(nrm, np.float32(np.sqrt(LN_EPS)))
    v1 = [v_in[c] / nrm for c in range(3)]

    vh = [v1[c] @ w1h[...] for c in range(3)]
    sh = jnp.maximum(jnp.sqrt(vh[0] ** 2 + vh[1] ** 2 + vh[2] ** 2), EPS)
    sm = s1 @ w1ms[...] + sh @ w1mh[...] + b1[...]
    f_s = jnp.maximum(sm, 0.0)
    vmu = [vh[c] @ w1mu[...] for c in range(3)]
    nmu = jnp.maximum(jnp.sqrt(vmu[0] ** 2 + vmu[1] ** 2 + vmu[2] ** 2), EPS)
    gate = jax.nn.sigmoid(nmu)
    f_v = [gate * vmu[c] for c in range(3)]

    vh = [f_v[c] @ w2h[...] for c in range(3)]
    sh = jnp.maximum(jnp.sqrt(vh[0] ** 2 + vh[1] ** 2 + vh[2] ** 2), EPS)
    f_s = f_s @ w2ms[...] + sh @ w2mh[...] + b2[...]
    vmu = [vh[c] @ w2mu[...] for c in range(3)]
    nmu = jnp.maximum(jnp.sqrt(vmu[0] ** 2 + vmu[1] ** 2 + vmu[2] ** 2), EPS)
    f_v = [nmu * vmu[c] for c in range(3)]

    s_in = s1 + f_s
    mu = jnp.mean(s_in, axis=-1, keepdims=True)
    var = jnp.mean((s_in - mu) ** 2, axis=-1, keepdims=True)
    s_out[...] = (s_in - mu) * lax.rsqrt(var + LN_EPS) * ln2g[...] + ln2b[...]

    v_in = [v1[c] + f_v[c] for c in range(3)]
    nrm = (v_in[0] ** 2 + v_in[1] ** 2 + v_in[2] ** 2)
    nrm = jnp.sqrt(jnp.sum(nrm, axis=-1, keepdims=True) / NV)
    nrm = jnp.maximum(nrm, np.float32(np.sqrt(LN_EPS)))
    for c in range(3):
        v_out[:, 16 * c:16 * (c + 1)] = v_in[c] / nrm


def _node_compute(s, v48, a0, a1, params):
    w1h, w1mu, w1m, b1 = params['f1']
    w2h, w2mu, w2m, b2 = params['f2']
    wts = (
        params['ln1_g'].reshape(1, NS), params['ln1_b'].reshape(1, NS),
        params['ln2_g'].reshape(1, NS), params['ln2_b'].reshape(1, NS),
        jnp.asarray(w1h.T), jnp.asarray(w1mu.T),
        jnp.asarray(w1m[:, 0:NS].T), jnp.asarray(w1m[:, NS:].T), b1.reshape(1, NS),
        jnp.asarray(w2h.T), jnp.asarray(w2mu.T),
        jnp.asarray(w2m[:, 0:NS].T), jnp.asarray(w2m[:, NS:].T), b2.reshape(1, NS),
    )
    w_specs = [_full(w.shape) for w in wts]
    return pl.pallas_call(
        _node_kernel,
        grid=(N // NBLK,),
        in_specs=[_rows(NBLK, NS), _rows(NBLK, 3 * NV), _rows(NBLK, NS),
                  _rows(NBLK, NS), *w_specs],
        out_specs=[_rows(NBLK, NS), _rows(NBLK, 3 * NV)],
        out_shape=[jax.ShapeDtypeStruct((N, NS), jnp.float32),
                   jax.ShapeDtypeStruct((N, 3 * NV), jnp.float32)],
    )(s, v48, a0, a1, *wts)


def kernel(s, V, edge_index, edge_s, edge_V, params):
    src = edge_index[0]
    dst = edge_index[1]
    v48 = V.transpose(0, 2, 1).reshape(N, 3 * NV)
    ev3 = edge_V.reshape(E, 3)
    tab = jnp.concatenate(
        [s, v48, jnp.zeros((N, DP - D), jnp.float32)], axis=-1)
    dst2 = dst.reshape(_NW, _NCH, _GC)
    src2 = src.reshape(_NW, _NCH, _GC)
    dst16 = dst.reshape(16, _SNCH, _GC)

    td, ts = _sc_gather(tab, dst2, src2)
    msg_s, msg_v = _edge_compute(td, ts, edge_s, ev3, params)
    zeros = jnp.zeros((N, NS), jnp.float32)
    a0, a1 = _sc_scatter(msg_s, msg_v, dst16, zeros)

    s2, v2 = _node_compute(s, v48, a0, a1, params)
    return s2, v2.reshape(N, 3, NV).transpose(0, 2, 1)


# speedup vs baseline: 2.9346x; 1.1890x over previous
import functools

import jax
import jax.numpy as jnp
import numpy as np
from jax import lax
from jax.experimental import pallas as pl
from jax.experimental.pallas import tpu as pltpu
from jax.experimental.pallas import tpu_sc as plsc

N = 10000
E = 320000
NS, NV = 128, 16
ES, EV = 16, 1
EPS = 1e-4
LN_EPS = 1e-5

D = NS + 3 * NV
EBLK = 2000
NBLK = 1000

_NW = 32
_EPW = E // _NW
_GC = 80
_NCH = _EPW // _GC


DP = 256
_BF = jnp.bfloat16


def _gather_body(tab, dst2, src2, gd, gs, idxd, idxs, bufd, bufs, sem1, sem2):
    cid = lax.axis_index("c")
    sid = lax.axis_index("s")
    wid = sid * 2 + cid
    pltpu.sync_copy(dst2.at[wid], idxd)
    pltpu.sync_copy(src2.at[wid], idxs)

    def step(j, carry):
        eb = wid * _EPW + j * _GC
        cp1 = pltpu.async_copy(tab.at[idxd.at[j]], bufd, sem1)
        cp2 = pltpu.async_copy(tab.at[idxs.at[j]], bufs, sem2)
        cp1.wait()
        cp2.wait()
        pltpu.sync_copy(bufd, gd.at[pl.ds(eb, _GC)])
        pltpu.sync_copy(bufs, gs.at[pl.ds(eb, _GC)])
        return carry

    lax.fori_loop(0, _NCH, step, 0)


def _sc_gather(tab, dst2, src2):
    f = pl.kernel(
        _gather_body,
        out_type=[jax.ShapeDtypeStruct((E, DP), jnp.float32),
                  jax.ShapeDtypeStruct((E, DP), jnp.float32)],
        mesh=plsc.VectorSubcoreMesh(core_axis_name="c", subcore_axis_name="s"),
        scratch_types=[
            pltpu.VMEM((_NCH, _GC), jnp.int32),
            pltpu.VMEM((_NCH, _GC), jnp.int32),
            pltpu.VMEM((_GC, DP), jnp.float32),
            pltpu.VMEM((_GC, DP), jnp.float32),
            pltpu.SemaphoreType.DMA,
            pltpu.SemaphoreType.DMA,
        ],
    )
    return f(tab, dst2, src2)


_SNCH = (E // 16) // _GC


def _scatter_body(msg_s, msg_v, dst2, zeros, out_s, out_v, idx, buf, acc):
    cid = lax.axis_index("c")
    sid = lax.axis_index("s")

    @pl.when(sid == 0)
    def _init():
        pltpu.sync_copy(zeros, acc)

    pltpu.sync_copy(dst2.at[sid], idx)
    plsc.subcore_barrier()

    @pl.when(cid == 0)
    def _scat_s():
        def step(j, carry):
            eb = sid * (E // 16) + j * _GC
            pltpu.sync_copy(msg_s.at[pl.ds(eb, _GC)], buf)
            pltpu.sync_copy(buf, acc.at[idx.at[j]], add=True)
            return carry
        lax.fori_loop(0, _SNCH, step, 0)

    @pl.when(cid == 1)
    def _scat_v():
        def step(j, carry):
            eb = sid * (E // 16) + j * _GC
            pltpu.sync_copy(msg_v.at[pl.ds(eb, _GC)], buf)
            pltpu.sync_copy(buf, acc.at[idx.at[j]], add=True)
            return carry
        lax.fori_loop(0, _SNCH, step, 0)

    plsc.subcore_barrier()

    @pl.when((sid == 0) & (cid == 0))
    def _outs():
        pltpu.sync_copy(acc, out_s)

    @pl.when((sid == 0) & (cid == 1))
    def _outv():
        pltpu.sync_copy(acc, out_v)


def _sc_scatter(msg_s, msg_v, dst2, zeros):
    f = pl.kernel(
        _scatter_body,
        out_type=[jax.ShapeDtypeStruct((N, NS), jnp.float32),
                  jax.ShapeDtypeStruct((N, NS), jnp.float32)],
        mesh=plsc.VectorSubcoreMesh(core_axis_name="c", subcore_axis_name="s"),
        scratch_types=[
            pltpu.VMEM((_SNCH, _GC), jnp.int32),
            pltpu.VMEM((_GC, NS), jnp.float32),
            pltpu.VMEM_SHARED((N, NS), jnp.float32),
        ],
    )
    return f(msg_s, msg_v, dst2, zeros)


def _edge_kernel(td, ts, es, ev,
                 bdhd, bdhs, bdhe, bd1mu, g99, g48, w1md, w1ms, w1me, w1mh, b1,
                 bd2h, bd2mu, w2ms, w2mh, b2,
                 bd3h, bd3mu, w3ms, w3mh, b3,
                 bdah, wams, wamh, ba,
                 msg_s, msg_v):
    sd = td[:, 0:NS]
    ss = ts[:, 0:NS]
    vd = td[:, NS:NS + 48]
    vs = ts[:, NS:NS + 48]
    vh = vd @ bdhd[...] + vs @ bdhs[...] + ev[...] @ bdhe[...]
    sh = jnp.sqrt(jnp.maximum((vh * vh) @ g99[...], EPS * EPS))
    sm = (sd @ w1md[...] + ss @ w1ms[...]
          + es[...] @ w1me[...] + sh @ w1mh[...] + b1[...])
    m_s = jnp.maximum(sm, 0.0)
    vmu = vh @ bd1mu[...]
    nmu = jnp.sqrt(jnp.maximum((vmu * vmu) @ g48[...], EPS * EPS))
    gate = jax.nn.sigmoid(nmu)
    m_v = jnp.concatenate([gate, gate, gate], axis=-1) * vmu

    vh = m_v @ bd2h[...]
    sh = jnp.sqrt(jnp.maximum((vh * vh) @ g48[...], EPS * EPS))
    m_s = jnp.maximum(m_s @ w2ms[...] + sh @ w2mh[...] + b2[...], 0.0)
    vmu = vh @ bd2mu[...]
    nmu = jnp.sqrt(jnp.maximum((vmu * vmu) @ g48[...], EPS * EPS))
    gate = jax.nn.sigmoid(nmu)
    m_v = jnp.concatenate([gate, gate, gate], axis=-1) * vmu

    vh = m_v @ bd3h[...]
    sh = jnp.sqrt(jnp.maximum((vh * vh) @ g48[...], EPS * EPS))
    m_s = m_s @ w3ms[...] + sh @ w3mh[...] + b3[...]
    vmu = vh @ bd3mu[...]
    nmu = jnp.sqrt(jnp.maximum((vmu * vmu) @ g48[...], EPS * EPS))
    m_v = jnp.concatenate([nmu, nmu, nmu], axis=-1) * vmu

    vh = m_v @ bdah[...]
    sh = jnp.sqrt(jnp.maximum((vh * vh) @ g48[...], EPS * EPS))
    att = jax.nn.sigmoid(m_s @ wams[...] + sh @ wamh[...] + ba[...])

    msg_s[...] = att * m_s
    msg_v[:, 0:48] = att * m_v
    msg_v[:, 48:NS] = jnp.zeros((msg_v.shape[0], NS - 48), jnp.float32)


def _full(shape):
    return pl.BlockSpec(shape, lambda i: (0,) * len(shape))


def _rows(blk, cols):
    return pl.BlockSpec((blk, cols), lambda i: (i, 0))


def _edge_weights(params):
    w1h, w1mu, w1m, b1 = params['e1']
    w2h, w2mu, w2m, b2 = params['e2']
    w3h, w3mu, w3m, b3 = params['e3']
    wah, _, wam, ba = params['att']
    bd = lambda w: jax.scipy.linalg.block_diag(w, w, w)
    eye33 = jnp.eye(33, dtype=jnp.float32)
    eye16 = jnp.eye(16, dtype=jnp.float32)
    return (
        bd(w1h[:, 0:NV].T),
        bd(w1h[:, NV:2 * NV].T),
        bd(w1h[:, 2 * NV:].T),
        bd(w1mu.T),
        jnp.concatenate([eye33] * 3, axis=0),
        jnp.concatenate([eye16] * 3, axis=0),
        jnp.asarray(w1m[:, 0:NS].T),
        jnp.asarray(w1m[:, NS:2 * NS].T),
        jnp.asarray(w1m[:, 2 * NS:2 * NS + ES].T),
        jnp.asarray(w1m[:, 2 * NS + ES:].T),
        b1.reshape(1, NS),
        bd(w2h.T), bd(w2mu.T),
        jnp.asarray(w2m[:, 0:NS].T), jnp.asarray(w2m[:, NS:].T), b2.reshape(1, NS),
        bd(w3h.T), bd(w3mu.T),
        jnp.asarray(w3m[:, 0:NS].T), jnp.asarray(w3m[:, NS:].T), b3.reshape(1, NS),
        bd(wah.T),
        jnp.asarray(wam[:, 0:NS].T), jnp.asarray(wam[:, NS:].T), ba.reshape(1, 1),
    )


def _edge_compute(td, ts, es, ev3, params):
    wts = _edge_weights(params)
    w_specs = [_full(w.shape) for w in wts]
    return pl.pallas_call(
        _edge_kernel,
        grid=(E // EBLK,),
        in_specs=[_rows(EBLK, DP), _rows(EBLK, DP), _rows(EBLK, ES),
                  _rows(EBLK, 3), *w_specs],
        out_specs=[_rows(EBLK, NS), _rows(EBLK, NS)],
        out_shape=[jax.ShapeDtypeStruct((E, NS), jnp.float32),
                   jax.ShapeDtypeStruct((E, NS), jnp.float32)],
    )(td, ts, es, ev3, *wts)


def _node_kernel(s, v, a0, a1,
                 ln1g, ln1b, ln2g, ln2b,
                 w1h, w1mu, w1ms, w1mh, b1,
                 w2h, w2mu, w2ms, w2mh, b2,
                 s_out, v_out):
    s_in = s[...] + a0[...]
    v_in = [v[:, 16 * c:16 * (c + 1)] + a1[:, 16 * c:16 * (c + 1)]
            for c in range(3)]

    mu = jnp.mean(s_in, axis=-1, keepdims=True)
    var = jnp.mean((s_in - mu) ** 2, axis=-1, keepdims=True)
    s1 = (s_in - mu) * lax.rsqrt(var + LN_EPS) * ln1g[...] + ln1b[...]
    nrm = (v_in[0] ** 2 + v_in[1] ** 2 + v_in[2] ** 2)
    nrm = jnp.sqrt(jnp.sum(nrm, axis=-1, keepdims=True) / NV)
    nrm = jnp.maximum(nrm, np.float32(np.sqrt(LN_EPS)))
    v1 = [v_in[c] / nrm for c in range(3)]

    vh = [v1[c] @ w1h[...] for c in range(3)]
    sh = jnp.maximum(jnp.sqrt(vh[0] ** 2 + vh[1] ** 2 + vh[2] ** 2), EPS)
    sm = s1 @ w1ms[...] + sh @ w1mh[...] + b1[...]
    f_s = jnp.maximum(sm, 0.0)
    vmu = [vh[c] @ w1mu[...] for c in range(3)]
    nmu = jnp.maximum(jnp.sqrt(vmu[0] ** 2 + vmu[1] ** 2 + vmu[2] ** 2), EPS)
    gate = jax.nn.sigmoid(nmu)
    f_v = [gate * vmu[c] for c in range(3)]

    vh = [f_v[c] @ w2h[...] for c in range(3)]
    sh = jnp.maximum(jnp.sqrt(vh[0] ** 2 + vh[1] ** 2 + vh[2] ** 2), EPS)
    f_s = f_s @ w2ms[...] + sh @ w2mh[...] + b2[...]
    vmu = [vh[c] @ w2mu[...] for c in range(3)]
    nmu = jnp.maximum(jnp.sqrt(vmu[0] ** 2 + vmu[1] ** 2 + vmu[2] ** 2), EPS)
    f_v = [nmu * vmu[c] for c in range(3)]

    s_in = s1 + f_s
    mu = jnp.mean(s_in, axis=-1, keepdims=True)
    var = jnp.mean((s_in - mu) ** 2, axis=-1, keepdims=True)
    s_out[...] = (s_in - mu) * lax.rsqrt(var + LN_EPS) * ln2g[...] + ln2b[...]

    v_in = [v1[c] + f_v[c] for c in range(3)]
    nrm = (v_in[0] ** 2 + v_in[1] ** 2 + v_in[2] ** 2)
    nrm = jnp.sqrt(jnp.sum(nrm, axis=-1, keepdims=True) / NV)
    nrm = jnp.maximum(nrm, np.float32(np.sqrt(LN_EPS)))
    for c in range(3):
        v_out[:, 16 * c:16 * (c + 1)] = v_in[c] / nrm


def _node_compute(s, v48, a0, a1, params):
    w1h, w1mu, w1m, b1 = params['f1']
    w2h, w2mu, w2m, b2 = params['f2']
    wts = (
        params['ln1_g'].reshape(1, NS), params['ln1_b'].reshape(1, NS),
        params['ln2_g'].reshape(1, NS), params['ln2_b'].reshape(1, NS),
        jnp.asarray(w1h.T), jnp.asarray(w1mu.T),
        jnp.asarray(w1m[:, 0:NS].T), jnp.asarray(w1m[:, NS:].T), b1.reshape(1, NS),
        jnp.asarray(w2h.T), jnp.asarray(w2mu.T),
        jnp.asarray(w2m[:, 0:NS].T), jnp.asarray(w2m[:, NS:].T), b2.reshape(1, NS),
    )
    w_specs = [_full(w.shape) for w in wts]
    return pl.pallas_call(
        _node_kernel,
        grid=(N // NBLK,),
        in_specs=[_rows(NBLK, NS), _rows(NBLK, 3 * NV), _rows(NBLK, NS),
                  _rows(NBLK, NS), *w_specs],
        out_specs=[_rows(NBLK, NS), _rows(NBLK, 3 * NV)],
        out_shape=[jax.ShapeDtypeStruct((N, NS), jnp.float32),
                   jax.ShapeDtypeStruct((N, 3 * NV), jnp.float32)],
    )(s, v48, a0, a1, *wts)


def kernel(s, V, edge_index, edge_s, edge_V, params):
    src = edge_index[0]
    dst = edge_index[1]
    v48 = V.transpose(0, 2, 1).reshape(N, 3 * NV)
    ev3 = edge_V.reshape(E, 3)
    tab = jnp.concatenate(
        [s, v48, jnp.zeros((N, DP - D), jnp.float32)], axis=-1)
    dst2 = dst.reshape(_NW, _NCH, _GC)
    src2 = src.reshape(_NW, _NCH, _GC)
    dst16 = dst.reshape(16, _SNCH, _GC)

    td, ts = _sc_gather(tab, dst2, src2)
    msg_s, msg_v = _edge_compute(td, ts, edge_s, ev3, params)
    zeros = jnp.zeros((N, NS), jnp.float32)
    a0, a1 = _sc_scatter(msg_s, msg_v, dst16, zeros)

    s2, v2 = _node_compute(s, v48, a0, a1, params)
    return s2, v2.reshape(N, 3, NV).transpose(0, 2, 1)


# speedup vs baseline: 3.3423x; 1.1389x over previous
import functools

import jax
import jax.numpy as jnp
import numpy as np
from jax import lax
from jax.experimental import pallas as pl
from jax.experimental.pallas import tpu as pltpu
from jax.experimental.pallas import tpu_sc as plsc

N = 10000
E = 320000
NS, NV = 128, 16
ES, EV = 16, 1
EPS = 1e-4
LN_EPS = 1e-5

D = NS + 3 * NV
EBLK = 2000
NBLK = 1000

_NW = 32
_EPW = E // _NW
_GC = 80
_NCH = _EPW // _GC


DP = 256
_BF = jnp.bfloat16


def _gather_body(tab, dst2, src2, gd, gs, idxd, idxs, bufd, bufs, sem1, sem2):
    cid = lax.axis_index("c")
    sid = lax.axis_index("s")
    wid = sid * 2 + cid
    pltpu.sync_copy(dst2.at[wid], idxd)
    pltpu.sync_copy(src2.at[wid], idxs)

    def step(j, carry):
        eb = wid * _EPW + j * _GC
        cp1 = pltpu.async_copy(tab.at[idxd.at[j]], bufd, sem1)
        cp2 = pltpu.async_copy(tab.at[idxs.at[j]], bufs, sem2)
        cp1.wait()
        cp2.wait()
        pltpu.sync_copy(bufd, gd.at[pl.ds(eb, _GC)])
        pltpu.sync_copy(bufs, gs.at[pl.ds(eb, _GC)])
        return carry

    lax.fori_loop(0, _NCH, step, 0)


def _sc_gather(tab, dst2, src2):
    f = pl.kernel(
        _gather_body,
        out_type=[jax.ShapeDtypeStruct((E, NS), jnp.int32),
                  jax.ShapeDtypeStruct((E, NS), jnp.int32)],
        mesh=plsc.VectorSubcoreMesh(core_axis_name="c", subcore_axis_name="s"),
        scratch_types=[
            pltpu.VMEM((_NCH, _GC), jnp.int32),
            pltpu.VMEM((_NCH, _GC), jnp.int32),
            pltpu.VMEM((_GC, NS), jnp.int32),
            pltpu.VMEM((_GC, NS), jnp.int32),
            pltpu.SemaphoreType.DMA,
            pltpu.SemaphoreType.DMA,
        ],
    )
    return f(tab, dst2, src2)


_SNCH = (E // 16) // _GC


def _scatter_body(msg_s, msg_v, dst2, zeros, out_s, out_v, idx, buf, acc):
    cid = lax.axis_index("c")
    sid = lax.axis_index("s")

    @pl.when(sid == 0)
    def _init():
        pltpu.sync_copy(zeros, acc)

    pltpu.sync_copy(dst2.at[sid], idx)
    plsc.subcore_barrier()

    @pl.when(cid == 0)
    def _scat_s():
        def step(j, carry):
            eb = sid * (E // 16) + j * _GC
            pltpu.sync_copy(msg_s.at[pl.ds(eb, _GC)], buf)
            pltpu.sync_copy(buf, acc.at[idx.at[j]], add=True)
            return carry
        lax.fori_loop(0, _SNCH, step, 0)

    @pl.when(cid == 1)
    def _scat_v():
        def step(j, carry):
            eb = sid * (E // 16) + j * _GC
            pltpu.sync_copy(msg_v.at[pl.ds(eb, _GC)], buf)
            pltpu.sync_copy(buf, acc.at[idx.at[j]], add=True)
            return carry
        lax.fori_loop(0, _SNCH, step, 0)

    plsc.subcore_barrier()

    @pl.when((sid == 0) & (cid == 0))
    def _outs():
        pltpu.sync_copy(acc, out_s)

    @pl.when((sid == 0) & (cid == 1))
    def _outv():
        pltpu.sync_copy(acc, out_v)


def _sc_scatter(msg_s, msg_v, dst2, zeros):
    f = pl.kernel(
        _scatter_body,
        out_type=[jax.ShapeDtypeStruct((N, NS), jnp.float32),
                  jax.ShapeDtypeStruct((N, NS), jnp.float32)],
        mesh=plsc.VectorSubcoreMesh(core_axis_name="c", subcore_axis_name="s"),
        scratch_types=[
            pltpu.VMEM((_SNCH, _GC), jnp.int32),
            pltpu.VMEM((_GC, NS), jnp.float32),
            pltpu.VMEM_SHARED((N, NS), jnp.float32),
        ],
    )
    return f(msg_s, msg_v, dst2, zeros)


def _mm(a, b):
    return lax.dot_general(a, b[...], (((1,), (0,)), ((), ())),
                           preferred_element_type=jnp.float32)


def _edge_kernel(td, ts, es, ev,
                 bdhd, bdhs, bdhe, bd1mu, g99, g48, w1md, w1ms, w1me, w1mh, b1,
                 bd2h, bd2mu, w2ms, w2mh, b2,
                 bd3h, bd3mu, w3ms, w3mh, b3,
                 bdah, wams, wamh, ba,
                 msg_s, msg_v):
    wd = td[...]
    ws = ts[...]
    himask = jnp.int32(-65536)
    sd = lax.bitcast_convert_type(lax.shift_left(wd, 16), jnp.float32)
    ss = lax.bitcast_convert_type(lax.shift_left(ws, 16), jnp.float32)
    vd = lax.bitcast_convert_type(wd & himask, jnp.float32)[:, 0:48]
    vs = lax.bitcast_convert_type(ws & himask, jnp.float32)[:, 0:48]
    vh = vd @ bdhd[...] + vs @ bdhs[...] + ev[...] @ bdhe[...]
    sh = jnp.sqrt(jnp.maximum((vh * vh) @ g99[...], EPS * EPS))
    sm = (sd @ w1md[...] + ss @ w1ms[...]
          + es[...] @ w1me[...] + sh @ w1mh[...] + b1[...])
    m_s = jnp.maximum(sm, 0.0)
    vmu = vh @ bd1mu[...]
    nmu = jnp.sqrt(jnp.maximum((vmu * vmu) @ g48[...], EPS * EPS))
    gate = jax.nn.sigmoid(nmu)
    m_v = jnp.concatenate([gate, gate, gate], axis=-1) * vmu

    vh = m_v @ bd2h[...]
    sh = jnp.sqrt(jnp.maximum((vh * vh) @ g48[...], EPS * EPS))
    m_s = jnp.maximum(m_s @ w2ms[...] + sh @ w2mh[...] + b2[...], 0.0)
    vmu = vh @ bd2mu[...]
    nmu = jnp.sqrt(jnp.maximum((vmu * vmu) @ g48[...], EPS * EPS))
    gate = jax.nn.sigmoid(nmu)
    m_v = jnp.concatenate([gate, gate, gate], axis=-1) * vmu

    vh = m_v @ bd3h[...]
    sh = jnp.sqrt(jnp.maximum((vh * vh) @ g48[...], EPS * EPS))
    m_s = m_s @ w3ms[...] + sh @ w3mh[...] + b3[...]
    vmu = vh @ bd3mu[...]
    nmu = jnp.sqrt(jnp.maximum((vmu * vmu) @ g48[...], EPS * EPS))
    m_v = jnp.concatenate([nmu, nmu, nmu], axis=-1) * vmu

    vh = m_v @ bdah[...]
    sh = jnp.sqrt(jnp.maximum((vh * vh) @ g48[...], EPS * EPS))
    att = jax.nn.sigmoid(m_s @ wams[...] + sh @ wamh[...] + ba[...])

    msg_s[...] = att * m_s
    msg_v[:, 0:48] = att * m_v
    msg_v[:, 48:NS] = jnp.zeros((msg_v.shape[0], NS - 48), jnp.float32)


def _full(shape):
    return pl.BlockSpec(shape, lambda i: (0,) * len(shape))


def _rows(blk, cols):
    return pl.BlockSpec((blk, cols), lambda i: (i, 0))


def _edge_weights(params):
    w1h, w1mu, w1m, b1 = params['e1']
    w2h, w2mu, w2m, b2 = params['e2']
    w3h, w3mu, w3m, b3 = params['e3']
    wah, _, wam, ba = params['att']
    bd = lambda w: jax.scipy.linalg.block_diag(w, w, w)
    eye33 = jnp.eye(33, dtype=jnp.float32)
    eye16 = jnp.eye(16, dtype=jnp.float32)
    return (
        bd(w1h[:, 0:NV].T),
        bd(w1h[:, NV:2 * NV].T),
        bd(w1h[:, 2 * NV:].T),
        bd(w1mu.T),
        jnp.concatenate([eye33] * 3, axis=0),
        jnp.concatenate([eye16] * 3, axis=0),
        jnp.asarray(w1m[:, 0:NS].T),
        jnp.asarray(w1m[:, NS:2 * NS].T),
        jnp.asarray(w1m[:, 2 * NS:2 * NS + ES].T),
        jnp.asarray(w1m[:, 2 * NS + ES:].T),
        b1.reshape(1, NS),
        bd(w2h.T), bd(w2mu.T),
        jnp.asarray(w2m[:, 0:NS].T), jnp.asarray(w2m[:, NS:].T), b2.reshape(1, NS),
        bd(w3h.T), bd(w3mu.T),
        jnp.asarray(w3m[:, 0:NS].T), jnp.asarray(w3m[:, NS:].T), b3.reshape(1, NS),
        bd(wah.T),
        jnp.asarray(wam[:, 0:NS].T), jnp.asarray(wam[:, NS:].T), ba.reshape(1, 1),
    )


def _edge_compute(td, ts, es, ev3, params):
    wts = _edge_weights(params)
    w_specs = [_full(w.shape) for w in wts]
    return pl.pallas_call(
        _edge_kernel,
        grid=(E // EBLK,),
        in_specs=[_rows(EBLK, NS), _rows(EBLK, NS), _rows(EBLK, ES),
                  _rows(EBLK, 3), *w_specs],
        out_specs=[_rows(EBLK, NS), _rows(EBLK, NS)],
        out_shape=[jax.ShapeDtypeStruct((E, NS), jnp.float32),
                   jax.ShapeDtypeStruct((E, NS), jnp.float32)],
    )(td, ts, es, ev3, *wts)


def _node_kernel(s, v, a0, a1,
                 ln1g, ln1b, ln2g, ln2b,
                 w1h, w1mu, w1ms, w1mh, b1,
                 w2h, w2mu, w2ms, w2mh, b2,
                 s_out, v_out):
    s_in = s[...] + a0[...]
    v_in = [v[:, 16 * c:16 * (c + 1)] + a1[:, 16 * c:16 * (c + 1)]
            for c in range(3)]

    mu = jnp.mean(s_in, axis=-1, keepdims=True)
    var = jnp.mean((s_in - mu) ** 2, axis=-1, keepdims=True)
    s1 = (s_in - mu) * lax.rsqrt(var + LN_EPS) * ln1g[...] + ln1b[...]
    nrm = (v_in[0] ** 2 + v_in[1] ** 2 + v_in[2] ** 2)
    nrm = jnp.sqrt(jnp.sum(nrm, axis=-1, keepdims=True) / NV)
    nrm = jnp.maximum(nrm, np.float32(np.sqrt(LN_EPS)))
    v1 = [v_in[c] / nrm for c in range(3)]

    vh = [v1[c] @ w1h[...] for c in range(3)]
    sh = jnp.maximum(jnp.sqrt(vh[0] ** 2 + vh[1] ** 2 + vh[2] ** 2), EPS)
    sm = s1 @ w1ms[...] + sh @ w1mh[...] + b1[...]
    f_s = jnp.maximum(sm, 0.0)
    vmu = [vh[c] @ w1mu[...] for c in range(3)]
    nmu = jnp.maximum(jnp.sqrt(vmu[0] ** 2 + vmu[1] ** 2 + vmu[2] ** 2), EPS)
    gate = jax.nn.sigmoid(nmu)
    f_v = [gate * vmu[c] for c in range(3)]

    vh = [f_v[c] @ w2h[...] for c in range(3)]
    sh = jnp.maximum(jnp.sqrt(vh[0] ** 2 + vh[1] ** 2 + vh[2] ** 2), EPS)
    f_s = f_s @ w2ms[...] + sh @ w2mh[...] + b2[...]
    vmu = [vh[c] @ w2mu[...] for c in range(3)]
    nmu = jnp.maximum(jnp.sqrt(vmu[0] ** 2 + vmu[1] ** 2 + vmu[2] ** 2), EPS)
    f_v = [nmu * vmu[c] for c in range(3)]

    s_in = s1 + f_s
    mu = jnp.mean(s_in, axis=-1, keepdims=True)
    var = jnp.mean((s_in - mu) ** 2, axis=-1, keepdims=True)
    s_out[...] = (s_in - mu) * lax.rsqrt(var + LN_EPS) * ln2g[...] + ln2b[...]

    v_in = [v1[c] + f_v[c] for c in range(3)]
    nrm = (v_in[0] ** 2 + v_in[1] ** 2 + v_in[2] ** 2)
    nrm = jnp.sqrt(jnp.sum(nrm, axis=-1, keepdims=True) / NV)
    nrm = jnp.maximum(nrm, np.float32(np.sqrt(LN_EPS)))
    for c in range(3):
        v_out[:, 16 * c:16 * (c + 1)] = v_in[c] / nrm


def _node_compute(s, v48, a0, a1, params):
    w1h, w1mu, w1m, b1 = params['f1']
    w2h, w2mu, w2m, b2 = params['f2']
    wts = (
        params['ln1_g'].reshape(1, NS), params['ln1_b'].reshape(1, NS),
        params['ln2_g'].reshape(1, NS), params['ln2_b'].reshape(1, NS),
        jnp.asarray(w1h.T), jnp.asarray(w1mu.T),
        jnp.asarray(w1m[:, 0:NS].T), jnp.asarray(w1m[:, NS:].T), b1.reshape(1, NS),
        jnp.asarray(w2h.T), jnp.asarray(w2mu.T),
        jnp.asarray(w2m[:, 0:NS].T), jnp.asarray(w2m[:, NS:].T), b2.reshape(1, NS),
    )
    w_specs = [_full(w.shape) for w in wts]
    return pl.pallas_call(
        _node_kernel,
        grid=(N // NBLK,),
        in_specs=[_rows(NBLK, NS), _rows(NBLK, 3 * NV), _rows(NBLK, NS),
                  _rows(NBLK, NS), *w_specs],
        out_specs=[_rows(NBLK, NS), _rows(NBLK, 3 * NV)],
        out_shape=[jax.ShapeDtypeStruct((N, NS), jnp.float32),
                   jax.ShapeDtypeStruct((N, 3 * NV), jnp.float32)],
    )(s, v48, a0, a1, *wts)


def kernel(s, V, edge_index, edge_s, edge_V, params):
    src = edge_index[0]
    dst = edge_index[1]
    v48 = V.transpose(0, 2, 1).reshape(N, 3 * NV)
    ev3 = edge_V.reshape(E, 3)
    lo = lax.bitcast_convert_type(s.astype(_BF), jnp.uint16).astype(jnp.uint32)
    vpad = jnp.concatenate(
        [v48, jnp.zeros((N, NS - 48), jnp.float32)], axis=-1)
    hi = lax.bitcast_convert_type(vpad.astype(_BF), jnp.uint16).astype(jnp.uint32)
    tab = lax.bitcast_convert_type(lo | (hi << 16), jnp.int32)
    dst2 = dst.reshape(_NW, _NCH, _GC)
    src2 = src.reshape(_NW, _NCH, _GC)
    dst16 = dst.reshape(16, _SNCH, _GC)

    td, ts = _sc_gather(tab, dst2, src2)
    msg_s, msg_v = _edge_compute(td, ts, edge_s, ev3, params)
    zeros = jnp.zeros((N, NS), jnp.float32)
    a0, a1 = _sc_scatter(msg_s, msg_v, dst16, zeros)

    s2, v2 = _node_compute(s, v48, a0, a1, params)
    return s2, v2.reshape(N, 3, NV).transpose(0, 2, 1)


# speedup vs baseline: 3.6307x; 1.0863x over previous
import functools

import jax
import jax.numpy as jnp
import numpy as np
from jax import lax
from jax.experimental import pallas as pl
from jax.experimental.pallas import tpu as pltpu
from jax.experimental.pallas import tpu_sc as plsc

N = 10000
E = 320000
NS, NV = 128, 16
ES, EV = 16, 1
EPS = 1e-4
LN_EPS = 1e-5

D = NS + 3 * NV
EBLK = 2000
NBLK = 1000

_NW = 32
_EPW = E // _NW
_GC = 80
_NCH = _EPW // _GC


DP = 256
_BF = jnp.bfloat16


def _gather_body_for(epw, gc, nch):
    def body(tab, dst2, src2, gd, gs, idxd, idxs, bufd, bufs, sem1, sem2):
        cid = lax.axis_index("c")
        sid = lax.axis_index("s")
        wid = sid * 2 + cid
        pltpu.sync_copy(dst2.at[wid], idxd)
        pltpu.sync_copy(src2.at[wid], idxs)

        def step(j, carry):
            eb = wid * epw + j * gc
            cp1 = pltpu.async_copy(tab.at[idxd.at[j]], bufd, sem1)
            cp2 = pltpu.async_copy(tab.at[idxs.at[j]], bufs, sem2)
            cp1.wait()
            cp2.wait()
            pltpu.sync_copy(bufd, gd.at[pl.ds(eb, gc)])
            pltpu.sync_copy(bufs, gs.at[pl.ds(eb, gc)])
            return carry

        lax.fori_loop(0, nch, step, 0)
    return body


def _sc_gather(tab, dst2, src2, ne):
    epw = ne // _NW
    gc = 80 if epw % 80 == 0 else 40
    nch = epw // gc
    f = pl.kernel(
        _gather_body_for(epw, gc, nch),
        out_type=[jax.ShapeDtypeStruct((ne, NS), jnp.int32),
                  jax.ShapeDtypeStruct((ne, NS), jnp.int32)],
        mesh=plsc.VectorSubcoreMesh(core_axis_name="c", subcore_axis_name="s"),
        scratch_types=[
            pltpu.VMEM((nch, gc), jnp.int32),
            pltpu.VMEM((nch, gc), jnp.int32),
            pltpu.VMEM((gc, NS), jnp.int32),
            pltpu.VMEM((gc, NS), jnp.int32),
            pltpu.SemaphoreType.DMA,
            pltpu.SemaphoreType.DMA,
        ],
    )
    return f(tab, dst2, src2)


def _scatter_body_for(ept, snch):
    def body(msg_s, msg_v, dst2, zeros, out_s, out_v, idx, buf, acc):
        cid = lax.axis_index("c")
        sid = lax.axis_index("s")

        @pl.when(sid == 0)
        def _init():
            pltpu.sync_copy(zeros, acc)

        pltpu.sync_copy(dst2.at[sid], idx)
        plsc.subcore_barrier()

        @pl.when(cid == 0)
        def _scat_s():
            def step(j, carry):
                eb = sid * ept + j * _GC
                pltpu.sync_copy(msg_s.at[pl.ds(eb, _GC)], buf)
                pltpu.sync_copy(buf, acc.at[idx.at[j]], add=True)
                return carry
            lax.fori_loop(0, snch, step, 0)

        @pl.when(cid == 1)
        def _scat_v():
            def step(j, carry):
                eb = sid * ept + j * _GC
                pltpu.sync_copy(msg_v.at[pl.ds(eb, _GC)], buf)
                pltpu.sync_copy(buf, acc.at[idx.at[j]], add=True)
                return carry
            lax.fori_loop(0, snch, step, 0)

        plsc.subcore_barrier()

        @pl.when((sid == 0) & (cid == 0))
        def _outs():
            pltpu.sync_copy(acc, out_s)

        @pl.when((sid == 0) & (cid == 1))
        def _outv():
            pltpu.sync_copy(acc, out_v)
    return body


def _sc_scatter(msg_s, msg_v, dst2, zeros, ne):
    ept = ne // 16
    snch = ept // _GC
    f = pl.kernel(
        _scatter_body_for(ept, snch),
        out_type=[jax.ShapeDtypeStruct((N, NS), jnp.float32),
                  jax.ShapeDtypeStruct((N, NS), jnp.float32)],
        mesh=plsc.VectorSubcoreMesh(core_axis_name="c", subcore_axis_name="s"),
        scratch_types=[
            pltpu.VMEM((snch, _GC), jnp.int32),
            pltpu.VMEM((_GC, NS), jnp.float32),
            pltpu.VMEM_SHARED((N, NS), jnp.float32),
        ],
    )
    return f(msg_s, msg_v, dst2, zeros)


def _mm(a, b):
    return lax.dot_general(a, b[...], (((1,), (0,)), ((), ())),
                           preferred_element_type=jnp.float32)


def _edge_kernel(td, ts, es, ev,
                 bdhd, bdhs, bdhe, bd1mu, g99, g48, w1md, w1ms, w1me, w1mh, b1,
                 bd2h, bd2mu, w2ms, w2mh, b2,
                 bd3h, bd3mu, w3ms, w3mh, b3,
                 bdah, wams, wamh, ba,
                 msg_s, msg_v):
    wd = td[...]
    ws = ts[...]
    himask = jnp.int32(-65536)
    sd = lax.bitcast_convert_type(lax.shift_left(wd, 16), jnp.float32)
    ss = lax.bitcast_convert_type(lax.shift_left(ws, 16), jnp.float32)
    vd = lax.bitcast_convert_type(wd & himask, jnp.float32)[:, 0:48]
    vs = lax.bitcast_convert_type(ws & himask, jnp.float32)[:, 0:48]
    vh = vd @ bdhd[...] + vs @ bdhs[...] + ev[...] @ bdhe[...]
    sh = jnp.sqrt(jnp.maximum((vh * vh) @ g99[...], EPS * EPS))
    sm = (sd @ w1md[...] + ss @ w1ms[...]
          + es[...] @ w1me[...] + sh @ w1mh[...] + b1[...])
    m_s = jnp.maximum(sm, 0.0)
    vmu = vh @ bd1mu[...]
    nmu = jnp.sqrt(jnp.maximum((vmu * vmu) @ g48[...], EPS * EPS))
    gate = jax.nn.sigmoid(nmu)
    m_v = jnp.concatenate([gate, gate, gate], axis=-1) * vmu

    vh = m_v @ bd2h[...]
    sh = jnp.sqrt(jnp.maximum((vh * vh) @ g48[...], EPS * EPS))
    m_s = jnp.maximum(m_s @ w2ms[...] + sh @ w2mh[...] + b2[...], 0.0)
    vmu = vh @ bd2mu[...]
    nmu = jnp.sqrt(jnp.maximum((vmu * vmu) @ g48[...], EPS * EPS))
    gate = jax.nn.sigmoid(nmu)
    m_v = jnp.concatenate([gate, gate, gate], axis=-1) * vmu

    vh = m_v @ bd3h[...]
    sh = jnp.sqrt(jnp.maximum((vh * vh) @ g48[...], EPS * EPS))
    m_s = m_s @ w3ms[...] + sh @ w3mh[...] + b3[...]
    vmu = vh @ bd3mu[...]
    nmu = jnp.sqrt(jnp.maximum((vmu * vmu) @ g48[...], EPS * EPS))
    m_v = jnp.concatenate([nmu, nmu, nmu], axis=-1) * vmu

    vh = m_v @ bdah[...]
    sh = jnp.sqrt(jnp.maximum((vh * vh) @ g48[...], EPS * EPS))
    att = jax.nn.sigmoid(m_s @ wams[...] + sh @ wamh[...] + ba[...])

    msg_s[...] = att * m_s
    msg_v[:, 0:48] = att * m_v
    msg_v[:, 48:NS] = jnp.zeros((msg_v.shape[0], NS - 48), jnp.float32)


def _full(shape):
    return pl.BlockSpec(shape, lambda i: (0,) * len(shape))


def _rows(blk, cols):
    return pl.BlockSpec((blk, cols), lambda i: (i, 0))


def _edge_weights(params):
    w1h, w1mu, w1m, b1 = params['e1']
    w2h, w2mu, w2m, b2 = params['e2']
    w3h, w3mu, w3m, b3 = params['e3']
    wah, _, wam, ba = params['att']
    bd = lambda w: jax.scipy.linalg.block_diag(w, w, w)
    eye33 = jnp.eye(33, dtype=jnp.float32)
    eye16 = jnp.eye(16, dtype=jnp.float32)
    return (
        bd(w1h[:, 0:NV].T),
        bd(w1h[:, NV:2 * NV].T),
        bd(w1h[:, 2 * NV:].T),
        bd(w1mu.T),
        jnp.concatenate([eye33] * 3, axis=0),
        jnp.concatenate([eye16] * 3, axis=0),
        jnp.asarray(w1m[:, 0:NS].T),
        jnp.asarray(w1m[:, NS:2 * NS].T),
        jnp.asarray(w1m[:, 2 * NS:2 * NS + ES].T),
        jnp.asarray(w1m[:, 2 * NS + ES:].T),
        b1.reshape(1, NS),
        bd(w2h.T), bd(w2mu.T),
        jnp.asarray(w2m[:, 0:NS].T), jnp.asarray(w2m[:, NS:].T), b2.reshape(1, NS),
        bd(w3h.T), bd(w3mu.T),
        jnp.asarray(w3m[:, 0:NS].T), jnp.asarray(w3m[:, NS:].T), b3.reshape(1, NS),
        bd(wah.T),
        jnp.asarray(wam[:, 0:NS].T), jnp.asarray(wam[:, NS:].T), ba.reshape(1, 1),
    )


def _edge_compute(td, ts, es, ev3, wts, ne):
    w_specs = [_full(w.shape) for w in wts]
    return pl.pallas_call(
        _edge_kernel,
        grid=(ne // EBLK,),
        in_specs=[_rows(EBLK, NS), _rows(EBLK, NS), _rows(EBLK, ES),
                  _rows(EBLK, 3), *w_specs],
        out_specs=[_rows(EBLK, NS), _rows(EBLK, NS)],
        out_shape=[jax.ShapeDtypeStruct((ne, NS), jnp.float32),
                   jax.ShapeDtypeStruct((ne, NS), jnp.float32)],
    )(td, ts, es, ev3, *wts)


def _node_kernel(s, v, a0a, a0b, a1a, a1b,
                 ln1g, ln1b, ln2g, ln2b,
                 w1h, w1mu, w1ms, w1mh, b1,
                 w2h, w2mu, w2ms, w2mh, b2,
                 s_out, v_out):
    s_in = s[...] + a0a[...] + a0b[...]
    a1v = a1a[...] + a1b[...]
    v_in = [v[:, 16 * c:16 * (c + 1)] + a1v[:, 16 * c:16 * (c + 1)]
            for c in range(3)]

    mu = jnp.mean(s_in, axis=-1, keepdims=True)
    var = jnp.mean((s_in - mu) ** 2, axis=-1, keepdims=True)
    s1 = (s_in - mu) * lax.rsqrt(var + LN_EPS) * ln1g[...] + ln1b[...]
    nrm = (v_in[0] ** 2 + v_in[1] ** 2 + v_in[2] ** 2)
    nrm = jnp.sqrt(jnp.sum(nrm, axis=-1, keepdims=True) / NV)
    nrm = jnp.maximum(nrm, np.float32(np.sqrt(LN_EPS)))
    v1 = [v_in[c] / nrm for c in range(3)]

    vh = [v1[c] @ w1h[...] for c in range(3)]
    sh = jnp.maximum(jnp.sqrt(vh[0] ** 2 + vh[1] ** 2 + vh[2] ** 2), EPS)
    sm = s1 @ w1ms[...] + sh @ w1mh[...] + b1[...]
    f_s = jnp.maximum(sm, 0.0)
    vmu = [vh[c] @ w1mu[...] for c in range(3)]
    nmu = jnp.maximum(jnp.sqrt(vmu[0] ** 2 + vmu[1] ** 2 + vmu[2] ** 2), EPS)
    gate = jax.nn.sigmoid(nmu)
    f_v = [gate * vmu[c] for c in range(3)]

    vh = [f_v[c] @ w2h[...] for c in range(3)]
    sh = jnp.maximum(jnp.sqrt(vh[0] ** 2 + vh[1] ** 2 + vh[2] ** 2), EPS)
    f_s = f_s @ w2ms[...] + sh @ w2mh[...] + b2[...]
    vmu = [vh[c] @ w2mu[...] for c in range(3)]
    nmu = jnp.maximum(jnp.sqrt(vmu[0] ** 2 + vmu[1] ** 2 + vmu[2] ** 2), EPS)
    f_v = [nmu * vmu[c] for c in range(3)]

    s_in = s1 + f_s
    mu = jnp.mean(s_in, axis=-1, keepdims=True)
    var = jnp.mean((s_in - mu) ** 2, axis=-1, keepdims=True)
    s_out[...] = (s_in - mu) * lax.rsqrt(var + LN_EPS) * ln2g[...] + ln2b[...]

    v_in = [v1[c] + f_v[c] for c in range(3)]
    nrm = (v_in[0] ** 2 + v_in[1] ** 2 + v_in[2] ** 2)
    nrm = jnp.sqrt(jnp.sum(nrm, axis=-1, keepdims=True) / NV)
    nrm = jnp.maximum(nrm, np.float32(np.sqrt(LN_EPS)))
    for c in range(3):
        v_out[:, 16 * c:16 * (c + 1)] = v_in[c] / nrm


def _node_compute(s, v48, a0a, a0b, a1a, a1b, params):
    w1h, w1mu, w1m, b1 = params['f1']
    w2h, w2mu, w2m, b2 = params['f2']
    wts = (
        params['ln1_g'].reshape(1, NS), params['ln1_b'].reshape(1, NS),
        params['ln2_g'].reshape(1, NS), params['ln2_b'].reshape(1, NS),
        jnp.asarray(w1h.T), jnp.asarray(w1mu.T),
        jnp.asarray(w1m[:, 0:NS].T), jnp.asarray(w1m[:, NS:].T), b1.reshape(1, NS),
        jnp.asarray(w2h.T), jnp.asarray(w2mu.T),
        jnp.asarray(w2m[:, 0:NS].T), jnp.asarray(w2m[:, NS:].T), b2.reshape(1, NS),
    )
    w_specs = [_full(w.shape) for w in wts]
    return pl.pallas_call(
        _node_kernel,
        grid=(N // NBLK,),
        in_specs=[_rows(NBLK, NS), _rows(NBLK, 3 * NV), _rows(NBLK, NS),
                  _rows(NBLK, NS), _rows(NBLK, NS), _rows(NBLK, NS), *w_specs],
        out_specs=[_rows(NBLK, NS), _rows(NBLK, 3 * NV)],
        out_shape=[jax.ShapeDtypeStruct((N, NS), jnp.float32),
                   jax.ShapeDtypeStruct((N, 3 * NV), jnp.float32)],
    )(s, v48, a0a, a0b, a1a, a1b, *wts)


def kernel(s, V, edge_index, edge_s, edge_V, params):
    src = edge_index[0]
    dst = edge_index[1]
    v48 = V.transpose(0, 2, 1).reshape(N, 3 * NV)
    ev3 = edge_V.reshape(E, 3)
    lo = lax.bitcast_convert_type(s.astype(_BF), jnp.uint16).astype(jnp.uint32)
    vpad = jnp.concatenate(
        [v48, jnp.zeros((N, NS - 48), jnp.float32)], axis=-1)
    hi = lax.bitcast_convert_type(vpad.astype(_BF), jnp.uint16).astype(jnp.uint32)
    tab = lax.bitcast_convert_type(lo | (hi << 16), jnp.int32)
    eh = E // 2
    wts = _edge_weights(params)
    zeros = jnp.zeros((N, NS), jnp.float32)
    halves = []
    for h in range(2):
        sl = slice(h * eh, (h + 1) * eh)
        d = dst[sl]
        sr = src[sl]
        epw = eh // _NW
        gc = 80 if epw % 80 == 0 else 40
        td, ts = _sc_gather(tab, d.reshape(_NW, epw // gc, gc),
                            sr.reshape(_NW, epw // gc, gc), eh)
        msg_s, msg_v = _edge_compute(td, ts, edge_s[sl], ev3[sl], wts, eh)
        ept = eh // 16
        a0, a1 = _sc_scatter(msg_s, msg_v,
                             d.reshape(16, ept // _GC, _GC), zeros, eh)
        halves.append((a0, a1))

    (a0a, a1a), (a0b, a1b) = halves
    s2, v2 = _node_compute(s, v48, a0a, a0b, a1a, a1b, params)
    return s2, v2.reshape(N, 3, NV).transpose(0, 2, 1)
